# baseline (speedup 1.0000x reference)
import functools

import jax
import jax.numpy as jnp
from jax import lax
from jax.experimental import pallas as pl
from jax.experimental.pallas import tpu as pltpu
from jax.experimental.pallas import tpu_sc as plsc

NODES = 16384
H = 1.0 / (NODES - 1)
SCALE = float(NODES - 1)

NC, NS, L = 2, 16, 16
NW = NC * NS
N_TOTAL = 16777216
N_PER = N_TOTAL // NW
CHUNK = 8192
N_CHUNKS = N_PER // CHUNK
VECS = CHUNK // L


def _coef_body(y_ref, c_ref):
    y = y_ref[...]
    m = (y[:, 1:] - y[:, :-1]) / H
    left = jnp.concatenate(
        [3.0 * m[:, :1] - 2.0 * m[:, 1:2], 2.0 * m[:, :1] - m[:, 1:2]], axis=1)
    right = jnp.concatenate(
        [2.0 * m[:, -1:] - m[:, -2:-1], 3.0 * m[:, -1:] - 2.0 * m[:, -2:-1]],
        axis=1)
    mm = jnp.concatenate([left, m, right], axis=1)
    w1 = jnp.abs(mm[:, 3:] - mm[:, 2:-1])
    w2 = jnp.abs(mm[:, 1:-2] - mm[:, :-3])
    denom = w1 + w2
    safe = denom > 1e-9
    denom_safe = jnp.where(safe, denom, 1.0)
    t = jnp.where(safe, (w1 * mm[:, 1:-2] + w2 * mm[:, 2:-1]) / denom_safe,
                  0.5 * (mm[:, 1:-2] + mm[:, 2:-1]))
    t0 = t[:, :-1]
    t1 = t[:, 1:]
    c2 = (3.0 * m - 2.0 * t0 - t1) / H
    c3 = (t0 + t1 - 2.0 * m) / (H * H)
    pad = jnp.zeros((1, 1), jnp.float32)
    c_ref[0:1, :] = y
    c_ref[1:2, :] = t
    c_ref[2:3, :] = jnp.concatenate([c2, pad], axis=1)
    c_ref[3:4, :] = jnp.concatenate([c3, pad], axis=1)


def _coef_tables(value):
    return pl.pallas_call(
        _coef_body,
        out_shape=jax.ShapeDtypeStruct((4, NODES), jnp.float32),
    )(value.reshape(1, NODES))


def _sc_body(x_hbm, ctab_hbm, out_hbm, c0_v, c1_v, c2_v, c3_v,
             xb0, xb1, ob0, ob1, sem_in, sem_out, sem_tab):
    wid = lax.axis_index("s") * NC + lax.axis_index("c")
    base = wid * N_PER
    pltpu.async_copy(ctab_hbm.at[0], c0_v, sem_tab)
    pltpu.async_copy(ctab_hbm.at[1], c1_v, sem_tab)
    pltpu.async_copy(ctab_hbm.at[2], c2_v, sem_tab)
    pltpu.async_copy(ctab_hbm.at[3], c3_v, sem_tab)

    pltpu.async_copy(x_hbm.at[pl.ds(base, CHUNK)], xb0, sem_in)
    pltpu.async_copy(x_hbm.at[pl.ds(base + CHUNK, CHUNK)], xb1, sem_in)
    pltpu.make_async_copy(ctab_hbm.at[0], c0_v, sem_tab).wait()
    pltpu.make_async_copy(ctab_hbm.at[1], c1_v, sem_tab).wait()
    pltpu.make_async_copy(ctab_hbm.at[2], c2_v, sem_tab).wait()
    pltpu.make_async_copy(ctab_hbm.at[3], c3_v, sem_tab).wait()

    @pl.loop(0, N_CHUNKS, step=2)
    def _pair(g):
        for b in range(2):
            xb = (xb0, xb1)[b]
            ob = (ob0, ob1)[b]
            gg = g + b
            off = base + gg * CHUNK
            pltpu.make_async_copy(
                x_hbm.at[pl.ds(off, CHUNK)], xb, sem_in).wait()

            @pl.when(gg >= 2)
            def _wait_out():
                pltpu.make_async_copy(
                    ob, out_hbm.at[pl.ds(off - 2 * CHUNK, CHUNK)],
                    sem_out).wait()

            @plsc.parallel_loop(0, CHUNK, step=L, unroll=12)
            def _vec(e):
                xc = xb[pl.ds(e, L)]
                idx = jnp.minimum((xc * SCALE).astype(jnp.int32), NODES - 2)
                s = xc - idx.astype(jnp.float32) * H
                y0 = plsc.load_gather(c0_v, [idx])
                d0 = plsc.load_gather(c1_v, [idx])
                q2 = plsc.load_gather(c2_v, [idx])
                q3 = plsc.load_gather(c3_v, [idx])
                ob[pl.ds(e, L)] = y0 + s * (d0 + s * (q2 + s * q3))

            @pl.when(gg + 2 < N_CHUNKS)
            def _next_in():
                pltpu.async_copy(
                    x_hbm.at[pl.ds(off + 2 * CHUNK, CHUNK)], xb, sem_in)

            pltpu.async_copy(ob, out_hbm.at[pl.ds(off, CHUNK)], sem_out)

    pltpu.make_async_copy(
        ob0, out_hbm.at[pl.ds(base + (N_CHUNKS - 2) * CHUNK, CHUNK)],
        sem_out).wait()
    pltpu.make_async_copy(
        ob1, out_hbm.at[pl.ds(base + (N_CHUNKS - 1) * CHUNK, CHUNK)],
        sem_out).wait()


@functools.partial(jax.jit, static_argnames=())
def kernel(input, value):
    ctab = _coef_tables(value)
    mesh = plsc.VectorSubcoreMesh(core_axis_name="c", subcore_axis_name="s")
    akima = pl.kernel(
        _sc_body,
        out_type=jax.ShapeDtypeStruct((N_TOTAL,), jnp.float32),
        mesh=mesh,
        scratch_types=[
            pltpu.VMEM((NODES,), jnp.float32),
            pltpu.VMEM((NODES,), jnp.float32),
            pltpu.VMEM((NODES,), jnp.float32),
            pltpu.VMEM((NODES,), jnp.float32),
            pltpu.VMEM((CHUNK,), jnp.float32),
            pltpu.VMEM((CHUNK,), jnp.float32),
            pltpu.VMEM((CHUNK,), jnp.float32),
            pltpu.VMEM((CHUNK,), jnp.float32),
            pltpu.SemaphoreType.DMA,
            pltpu.SemaphoreType.DMA,
            pltpu.SemaphoreType.DMA,
        ],
        compiler_params=pltpu.CompilerParams(
            use_tc_tiling_on_sc=False, needs_layout_passes=False),
    )
    return akima(input, ctab)


# speedup vs baseline: 1.1024x; 1.1024x over previous
import functools

import jax
import jax.numpy as jnp
from jax import lax
from jax.experimental import pallas as pl
from jax.experimental.pallas import tpu as pltpu
from jax.experimental.pallas import tpu_sc as plsc

NODES = 16384
H = 1.0 / (NODES - 1)
SCALE = float(NODES - 1)

NC, NS, L = 2, 16, 16
NW = NC * NS
N_TOTAL = 16777216
N_PER = N_TOTAL // NW
CHUNK = 4096
N_CHUNKS = N_PER // CHUNK
VECS = CHUNK // L


def _coef_body(y_ref, c_ref):
    y = y_ref[...]
    m = (y[:, 1:] - y[:, :-1]) / H
    left = jnp.concatenate(
        [3.0 * m[:, :1] - 2.0 * m[:, 1:2], 2.0 * m[:, :1] - m[:, 1:2]], axis=1)
    right = jnp.concatenate(
        [2.0 * m[:, -1:] - m[:, -2:-1], 3.0 * m[:, -1:] - 2.0 * m[:, -2:-1]],
        axis=1)
    mm = jnp.concatenate([left, m, right], axis=1)
    w1 = jnp.abs(mm[:, 3:] - mm[:, 2:-1])
    w2 = jnp.abs(mm[:, 1:-2] - mm[:, :-3])
    denom = w1 + w2
    safe = denom > 1e-9
    denom_safe = jnp.where(safe, denom, 1.0)
    t = jnp.where(safe, (w1 * mm[:, 1:-2] + w2 * mm[:, 2:-1]) / denom_safe,
                  0.5 * (mm[:, 1:-2] + mm[:, 2:-1]))
    t0 = t[:, :-1]
    t1 = t[:, 1:]
    c2 = (3.0 * m - 2.0 * t0 - t1) / H
    c3 = (t0 + t1 - 2.0 * m) / (H * H)
    pad = jnp.zeros((1, 1), jnp.float32)
    c_ref[0:1, :] = y
    c_ref[1:2, :] = t
    c_ref[2:3, :] = jnp.concatenate([c2, pad], axis=1)
    c_ref[3:4, :] = jnp.concatenate([c3, pad], axis=1)


def _coef_tables(value):
    return pl.pallas_call(
        _coef_body,
        out_shape=jax.ShapeDtypeStruct((4, NODES), jnp.float32),
    )(value.reshape(1, NODES))


def _sc_body(x_hbm, ctab_hbm, out_hbm, c0_v, c1_v, c2_v, c3_v,
             xb0, xb1, ob0, ob1, sem_in, sem_out, sem_tab):
    wid = lax.axis_index("s") * NC + lax.axis_index("c")
    base = wid * N_PER
    pltpu.async_copy(ctab_hbm.at[0], c0_v, sem_tab)
    pltpu.async_copy(ctab_hbm.at[1], c1_v, sem_tab)
    pltpu.async_copy(ctab_hbm.at[2], c2_v, sem_tab)
    pltpu.async_copy(ctab_hbm.at[3], c3_v, sem_tab)

    pltpu.async_copy(x_hbm.at[pl.ds(base, CHUNK)], xb0, sem_in)
    pltpu.async_copy(x_hbm.at[pl.ds(base + CHUNK, CHUNK)], xb1, sem_in)
    pltpu.make_async_copy(ctab_hbm.at[0], c0_v, sem_tab).wait()
    pltpu.make_async_copy(ctab_hbm.at[1], c1_v, sem_tab).wait()
    pltpu.make_async_copy(ctab_hbm.at[2], c2_v, sem_tab).wait()
    pltpu.make_async_copy(ctab_hbm.at[3], c3_v, sem_tab).wait()

    @pl.loop(0, N_CHUNKS, step=2)
    def _pair(g):
        for b in range(2):
            xb = (xb0, xb1)[b]
            ob = (ob0, ob1)[b]
            gg = g + b
            off = base + gg * CHUNK
            pltpu.make_async_copy(
                x_hbm.at[pl.ds(off, CHUNK)], xb, sem_in).wait()

            @pl.when(gg >= 2)
            def _wait_out():
                pltpu.make_async_copy(
                    ob, out_hbm.at[pl.ds(off - 2 * CHUNK, CHUNK)],
                    sem_out).wait()

            @plsc.parallel_loop(0, CHUNK, step=L, unroll=8)
            def _vec(e):
                xc = xb[pl.ds(e, L)]
                idx = jnp.minimum((xc * SCALE).astype(jnp.int32), NODES - 2)
                s = xc - idx.astype(jnp.float32) * H
                y0 = plsc.load_gather(c0_v, [idx])
                d0 = plsc.load_gather(c1_v, [idx])
                q2 = plsc.load_gather(c2_v, [idx])
                q3 = plsc.load_gather(c3_v, [idx])
                ob[pl.ds(e, L)] = y0 + s * (d0 + s * (q2 + s * q3))

            @pl.when(gg + 2 < N_CHUNKS)
            def _next_in():
                pltpu.async_copy(
                    x_hbm.at[pl.ds(off + 2 * CHUNK, CHUNK)], xb, sem_in)

            pltpu.async_copy(ob, out_hbm.at[pl.ds(off, CHUNK)], sem_out)

    pltpu.make_async_copy(
        ob0, out_hbm.at[pl.ds(base + (N_CHUNKS - 2) * CHUNK, CHUNK)],
        sem_out).wait()
    pltpu.make_async_copy(
        ob1, out_hbm.at[pl.ds(base + (N_CHUNKS - 1) * CHUNK, CHUNK)],
        sem_out).wait()


@functools.partial(jax.jit, static_argnames=())
def kernel(input, value):
    ctab = _coef_tables(value)
    mesh = plsc.VectorSubcoreMesh(core_axis_name="c", subcore_axis_name="s")
    akima = pl.kernel(
        _sc_body,
        out_type=jax.ShapeDtypeStruct((N_TOTAL,), jnp.float32),
        mesh=mesh,
        scratch_types=[
            pltpu.VMEM((NODES,), jnp.float32),
            pltpu.VMEM((NODES,), jnp.float32),
            pltpu.VMEM((NODES,), jnp.float32),
            pltpu.VMEM((NODES,), jnp.float32),
            pltpu.VMEM((CHUNK,), jnp.float32),
            pltpu.VMEM((CHUNK,), jnp.float32),
            pltpu.VMEM((CHUNK,), jnp.float32),
            pltpu.VMEM((CHUNK,), jnp.float32),
            pltpu.SemaphoreType.DMA,
            pltpu.SemaphoreType.DMA,
            pltpu.SemaphoreType.DMA,
        ],
        compiler_params=pltpu.CompilerParams(
            use_tc_tiling_on_sc=False, needs_layout_passes=False),
    )
    return akima(input, ctab)


# speedup vs baseline: 1.1247x; 1.0202x over previous
import functools

import jax
import jax.numpy as jnp
from jax import lax
from jax.experimental import pallas as pl
from jax.experimental.pallas import tpu as pltpu
from jax.experimental.pallas import tpu_sc as plsc

NODES = 16384
H = 1.0 / (NODES - 1)
SCALE = float(NODES - 1)

NC, NS, L = 2, 16, 16
NW = NC * NS
N_TOTAL = 16777216
N_PER = N_TOTAL // NW
CHUNK = 8192
N_CHUNKS = N_PER // CHUNK
VECS = CHUNK // L


def _coef_body(y_ref, c_ref):
    y = y_ref[...]
    m = (y[:, 1:] - y[:, :-1]) / H
    left = jnp.concatenate(
        [3.0 * m[:, :1] - 2.0 * m[:, 1:2], 2.0 * m[:, :1] - m[:, 1:2]], axis=1)
    right = jnp.concatenate(
        [2.0 * m[:, -1:] - m[:, -2:-1], 3.0 * m[:, -1:] - 2.0 * m[:, -2:-1]],
        axis=1)
    mm = jnp.concatenate([left, m, right], axis=1)
    w1 = jnp.abs(mm[:, 3:] - mm[:, 2:-1])
    w2 = jnp.abs(mm[:, 1:-2] - mm[:, :-3])
    denom = w1 + w2
    safe = denom > 1e-9
    denom_safe = jnp.where(safe, denom, 1.0)
    t = jnp.where(safe, (w1 * mm[:, 1:-2] + w2 * mm[:, 2:-1]) / denom_safe,
                  0.5 * (mm[:, 1:-2] + mm[:, 2:-1]))
    t0 = t[:, :-1]
    t1 = t[:, 1:]
    c2 = (3.0 * m - 2.0 * t0 - t1) / H
    c3 = (t0 + t1 - 2.0 * m) / (H * H)
    pad = jnp.zeros((1, 1), jnp.float32)
    c_ref[0:1, :] = y
    c_ref[1:2, :] = t
    c_ref[2:3, :] = jnp.concatenate([c2, pad], axis=1)
    c_ref[3:4, :] = jnp.concatenate([c3, pad], axis=1)


def _coef_tables(value):
    return pl.pallas_call(
        _coef_body,
        out_shape=jax.ShapeDtypeStruct((4, NODES), jnp.float32),
    )(value.reshape(1, NODES))


def _sc_body(x_hbm, ctab_hbm, out_hbm, c0_v, c1_v, c2_v, c3_v,
             xb0, xb1, ob0, ob1, sem_in, sem_out, sem_tab):
    wid = lax.axis_index("s") * NC + lax.axis_index("c")
    base = wid * N_PER
    pltpu.async_copy(ctab_hbm.at[0], c0_v, sem_tab)
    pltpu.async_copy(ctab_hbm.at[1], c1_v, sem_tab)
    pltpu.async_copy(ctab_hbm.at[2], c2_v, sem_tab)
    pltpu.async_copy(ctab_hbm.at[3], c3_v, sem_tab)

    pltpu.async_copy(x_hbm.at[pl.ds(base, CHUNK)], xb0, sem_in)
    pltpu.async_copy(x_hbm.at[pl.ds(base + CHUNK, CHUNK)], xb1, sem_in)
    pltpu.make_async_copy(ctab_hbm.at[0], c0_v, sem_tab).wait()
    pltpu.make_async_copy(ctab_hbm.at[1], c1_v, sem_tab).wait()
    pltpu.make_async_copy(ctab_hbm.at[2], c2_v, sem_tab).wait()
    pltpu.make_async_copy(ctab_hbm.at[3], c3_v, sem_tab).wait()

    @pl.loop(0, N_CHUNKS, step=2)
    def _pair(g):
        for b in range(2):
            xb = (xb0, xb1)[b]
            ob = (ob0, ob1)[b]
            gg = g + b
            off = base + gg * CHUNK
            pltpu.make_async_copy(
                x_hbm.at[pl.ds(off, CHUNK)], xb, sem_in).wait()

            @pl.when(gg >= 2)
            def _wait_out():
                pltpu.make_async_copy(
                    ob, out_hbm.at[pl.ds(off - 2 * CHUNK, CHUNK)],
                    sem_out).wait()

            @plsc.parallel_loop(0, CHUNK, step=L, unroll=8)
            def _vec(e):
                xc = xb[pl.ds(e, L)]
                idx = jnp.minimum((xc * SCALE).astype(jnp.int32), NODES - 2)
                s = xc - idx.astype(jnp.float32) * H
                y0 = plsc.load_gather(c0_v, [idx])
                d0 = plsc.load_gather(c1_v, [idx])
                q2 = plsc.load_gather(c2_v, [idx])
                q3 = plsc.load_gather(c3_v, [idx])
                ob[pl.ds(e, L)] = y0 + s * (d0 + s * (q2 + s * q3))

            @pl.when(gg + 2 < N_CHUNKS)
            def _next_in():
                pltpu.async_copy(
                    x_hbm.at[pl.ds(off + 2 * CHUNK, CHUNK)], xb, sem_in)

            pltpu.async_copy(ob, out_hbm.at[pl.ds(off, CHUNK)], sem_out)

    pltpu.make_async_copy(
        ob0, out_hbm.at[pl.ds(base + (N_CHUNKS - 2) * CHUNK, CHUNK)],
        sem_out).wait()
    pltpu.make_async_copy(
        ob1, out_hbm.at[pl.ds(base + (N_CHUNKS - 1) * CHUNK, CHUNK)],
        sem_out).wait()


@functools.partial(jax.jit, static_argnames=())
def kernel(input, value):
    ctab = _coef_tables(value)
    mesh = plsc.VectorSubcoreMesh(core_axis_name="c", subcore_axis_name="s")
    akima = pl.kernel(
        _sc_body,
        out_type=jax.ShapeDtypeStruct((N_TOTAL,), jnp.float32),
        mesh=mesh,
        scratch_types=[
            pltpu.VMEM((NODES,), jnp.float32),
            pltpu.VMEM((NODES,), jnp.float32),
            pltpu.VMEM((NODES,), jnp.float32),
            pltpu.VMEM((NODES,), jnp.float32),
            pltpu.VMEM((CHUNK,), jnp.float32),
            pltpu.VMEM((CHUNK,), jnp.float32),
            pltpu.VMEM((CHUNK,), jnp.float32),
            pltpu.VMEM((CHUNK,), jnp.float32),
            pltpu.SemaphoreType.DMA,
            pltpu.SemaphoreType.DMA,
            pltpu.SemaphoreType.DMA,
        ],
        compiler_params=pltpu.CompilerParams(
            use_tc_tiling_on_sc=False, needs_layout_passes=False),
    )
    return akima(input, ctab)
